# baseline (speedup 1.0000x reference)
import functools

import jax
import jax.numpy as jnp
from jax import lax
from jax.experimental import pallas as pl
from jax.experimental.pallas import tpu as pltpu
from jax.experimental.pallas import tpu_sc as plsc

NC, NS, L = 2, 16, 16
NW = NC * NS
B = 128
BAG = 64
BLK = 1024
_C0 = 304


def _mesh():
    return plsc.VectorSubcoreMesh(core_axis_name="c", subcore_axis_name="s")


def _sc_deg(dst2d, ew2d, n_pad):
    chunks = dst2d.shape[0]
    cpt = chunks // NW
    rpt = n_pad // NS

    @functools.partial(
        pl.kernel,
        out_type=jax.ShapeDtypeStruct((NC, n_pad), jnp.float32),
        mesh=_mesh(),
        scratch_types=[
            pltpu.VMEM((cpt, B), jnp.int32),
            pltpu.VMEM((cpt, B), jnp.float32),
            pltpu.VMEM((rpt,), jnp.float32),
            pltpu.VMEM_SHARED((n_pad,), jnp.float32),
        ],
    )
    def k(dst_hbm, ew_hbm, degp_hbm, dstb, ewb, zbuf, deg_sh):
        cid = lax.axis_index("c")
        sid = lax.axis_index("s")
        wid = cid * NS + sid

        def zb(i, carry):
            zbuf[pl.ds(i * L, L)] = jnp.zeros((L,), jnp.float32)
            return carry

        lax.fori_loop(0, rpt // L, zb, 0)
        pltpu.sync_copy(zbuf, deg_sh.at[pl.ds(sid * rpt, rpt)])
        pltpu.sync_copy(dst_hbm.at[pl.ds(wid * cpt, cpt)], dstb)
        pltpu.sync_copy(ew_hbm.at[pl.ds(wid * cpt, cpt)], ewb)
        plsc.subcore_barrier()

        def body(c, carry):
            pltpu.sync_copy(ewb.at[c], deg_sh.at[dstb.at[c]], add=True)
            return carry

        lax.fori_loop(0, cpt, body, 0)
        plsc.subcore_barrier()
        pltpu.sync_copy(deg_sh.at[pl.ds(sid * rpt, rpt)],
                        degp_hbm.at[cid, pl.ds(sid * rpt, rpt)])

    return k(dst2d, ew2d)


def _sc_agg(eidx, ewg, feat, n_pad, d, c0=None):
    chunks = eidx.shape[0]
    cpt = chunks // NW
    if c0 is None:
        c0 = cpt
    c1 = 2 * cpt - c0
    assert c0 % 4 == 0 and c1 % 4 == 0 and min(c0, c1) >= 8
    rpt = n_pad // NS
    nv = d // L

    @functools.partial(
        pl.kernel,
        out_type=jax.ShapeDtypeStruct((NC, n_pad, d), jnp.float32),
        mesh=_mesh(),
        compiler_params=pltpu.CompilerParams(use_tc_tiling_on_sc=False),
        scratch_types=[
            pltpu.VMEM((BAG, d), jnp.float32),
            pltpu.VMEM((BAG, d), jnp.float32),
            pltpu.VMEM((BAG, d), jnp.float32),
            pltpu.VMEM((BAG, d), jnp.float32),
            pltpu.VMEM((2, BAG), jnp.int32),
            pltpu.VMEM((2, BAG), jnp.int32),
            pltpu.VMEM((2, BAG), jnp.int32),
            pltpu.VMEM((2, BAG), jnp.int32),
            pltpu.VMEM((1, BAG), jnp.float32),
            pltpu.VMEM((1, BAG), jnp.float32),
            pltpu.VMEM((1, BAG), jnp.float32),
            pltpu.VMEM((1, BAG), jnp.float32),
            pltpu.VMEM_SHARED((n_pad, d), jnp.float32),
            pltpu.SemaphoreType.DMA,
            pltpu.SemaphoreType.DMA,
            pltpu.SemaphoreType.DMA,
            pltpu.SemaphoreType.DMA,
            pltpu.SemaphoreType.DMA,
            pltpu.SemaphoreType.DMA,
            pltpu.SemaphoreType.DMA,
            pltpu.SemaphoreType.DMA,
            pltpu.SemaphoreType.DMA,
            pltpu.SemaphoreType.DMA,
            pltpu.SemaphoreType.DMA,
            pltpu.SemaphoreType.DMA,
        ],
    )
    def k(ei_hbm, ew_hbm, feat_hbm, accp_hbm,
          rb0, rb1, sb0, sb1, ei0, ei1, ei2, ei3, ef0, ef1, ef2, ef3,
          acc_sh, gsem0, gsem1, ssem0, ssem1,
          isem0, isem1, isem2, isem3, fsem0, fsem1, fsem2, fsem3):
        cid = lax.axis_index("c")
        sid = lax.axis_index("s")
        base = jnp.where(cid == 0, sid * c0, NS * c0 + sid * c1)
        cnt = jnp.where(cid == 0, c0, c1)
        rbufs, sbufs = (rb0, rb1), (sb0, sb1)
        eibufs = (ei0, ei1, ei2, ei3)
        efbufs = (ef0, ef1, ef2, ef3)
        gsems, ssems = (gsem0, gsem1), (ssem0, ssem1)
        isems = (isem0, isem1, isem2, isem3)
        fsems = (fsem0, fsem1, fsem2, fsem3)

        def zr(r, carry):
            for j in range(nv):
                sb0[r, pl.ds(j * L, L)] = jnp.zeros((L,), jnp.float32)
            return carry

        lax.fori_loop(0, BAG, zr, 0)
        for j in range(rpt // BAG):
            pltpu.sync_copy(sb0, acc_sh.at[pl.ds(sid * rpt + j * BAG, BAG)])
        pltpu.sync_copy(ei_hbm.at[base], ei0)
        pltpu.sync_copy(ew_hbm.at[base], ef0)
        pltpu.async_copy(feat_hbm.at[ei0.at[0]], rb0, gsem0)
        pltpu.async_copy(ei_hbm.at[base + 1], ei1, isem1)
        pltpu.async_copy(ew_hbm.at[base + 1], ef1, fsem1)
        plsc.subcore_barrier()

        def body(g, carry):
            for b in range(4):
                c = 4 * g + b
                rb, sb = rbufs[b % 2], sbufs[b % 2]
                gsem, ssem = gsems[b % 2], ssems[b % 2]
                ein = eibufs[(b + 1) % 4]

                @pl.when(c + 1 < cnt)
                def _():
                    pltpu.make_async_copy(
                        ei_hbm.at[base + c + 1], ein, isems[(b + 1) % 4]
                    ).wait()
                    pltpu.async_copy(feat_hbm.at[ein.at[0]],
                                     rbufs[(b + 1) % 2], gsems[(b + 1) % 2])

                if b >= 2:
                    pltpu.make_async_copy(
                        sb, acc_sh.at[eibufs[b].at[1]], ssem).wait()
                else:
                    @pl.when(g > 0)
                    def _():
                        pltpu.make_async_copy(
                            sb, acc_sh.at[eibufs[b].at[1]], ssem).wait()

                @pl.when(c + 2 < cnt)
                def _():
                    pltpu.async_copy(ei_hbm.at[base + c + 2],
                                     eibufs[(b + 2) % 4], isems[(b + 2) % 4])
                    pltpu.async_copy(ew_hbm.at[base + c + 2],
                                     efbufs[(b + 2) % 4], fsems[(b + 2) % 4])

                pltpu.make_async_copy(
                    feat_hbm.at[eibufs[b].at[0]], rb, gsem).wait()
                if b != 0:
                    pltpu.make_async_copy(
                        ew_hbm.at[base + c], efbufs[b], fsems[b]).wait()
                else:
                    @pl.when(g > 0)
                    def _():
                        pltpu.make_async_copy(
                            ew_hbm.at[base + c], efbufs[b], fsems[b]).wait()

                def scale(gg, rc):
                    wv = efbufs[b][0, pl.ds(gg * L, L)]
                    for r in range(L):
                        w = wv[r]
                        row = gg * L + r
                        for j in range(nv):
                            sl = pl.ds(j * L, L)
                            sb[row, sl] = rb[row, sl] * w
                    return rc

                lax.fori_loop(0, BAG // L, scale, 0)
                pltpu.async_copy(sb, acc_sh.at[eibufs[b].at[1]], ssem,
                                 add=True)
            return carry

        lax.fori_loop(0, cnt // 4, body, 0)
        for b in range(2):
            pltpu.make_async_copy(
                sbufs[b], acc_sh.at[eibufs[b].at[1]], ssems[b]).wait()
        plsc.subcore_barrier()
        pltpu.sync_copy(acc_sh.at[pl.ds(sid * rpt, rpt)],
                        accp_hbm.at[cid, pl.ds(sid * rpt, rpt)])

    return k(eidx, ewg, feat)


def _tc_matmul(x, W):
    n, kdim = x.shape
    m = W.shape[1]

    def body(x_ref, w_ref, o_ref):
        o_ref[:] = jnp.dot(x_ref[:], w_ref[:],
                           preferred_element_type=jnp.float32)

    return pl.pallas_call(
        body,
        grid=(n // BLK,),
        in_specs=[pl.BlockSpec((BLK, kdim), lambda i: (i, 0)),
                  pl.BlockSpec((kdim, m), lambda i: (0, 0))],
        out_specs=pl.BlockSpec((BLK, m), lambda i: (i, 0)),
        out_shape=jax.ShapeDtypeStruct((n, m), jnp.float32),
    )(x, W)


def _tc_prescale(degpT, h):
    n, d = h.shape

    def body(dg_ref, h_ref, o_ref):
        deg = 1.0 + dg_ref[:, 0:1] + dg_ref[:, 1:2]
        dinv = lax.rsqrt(deg)
        o_ref[:] = h_ref[:] * dinv

    return pl.pallas_call(
        body,
        grid=(n // BLK,),
        in_specs=[pl.BlockSpec((BLK, 2), lambda i: (i, 0)),
                  pl.BlockSpec((BLK, d), lambda i: (i, 0))],
        out_specs=pl.BlockSpec((BLK, d), lambda i: (i, 0)),
        out_shape=jax.ShapeDtypeStruct((n, d), jnp.float32),
    )(degpT, h)


def _tc_mid(degpT, accp, h1, b1r, W2p):
    n, d1 = h1.shape
    d2 = W2p.shape[1]

    def body(dg_ref, ac_ref, h1_ref, b1_ref, w2_ref, h2_ref, hs2_ref):
        deg = 1.0 + dg_ref[:, 0:1] + dg_ref[:, 1:2]
        dinv = lax.rsqrt(deg)
        accsum = ac_ref[0] + ac_ref[1]
        x1 = dinv * accsum + (dinv * dinv) * h1_ref[:] + b1_ref[:]
        x1 = jnp.maximum(x1, 0.0)
        h2 = jnp.dot(x1, w2_ref[:], preferred_element_type=jnp.float32)
        h2_ref[:] = h2
        hs2_ref[:] = h2 * dinv

    return pl.pallas_call(
        body,
        grid=(n // BLK,),
        in_specs=[pl.BlockSpec((BLK, 2), lambda i: (i, 0)),
                  pl.BlockSpec((2, BLK, d1), lambda i: (0, i, 0)),
                  pl.BlockSpec((BLK, d1), lambda i: (i, 0)),
                  pl.BlockSpec((1, d1), lambda i: (0, 0)),
                  pl.BlockSpec((d1, d2), lambda i: (0, 0))],
        out_specs=[pl.BlockSpec((BLK, d2), lambda i: (i, 0)),
                   pl.BlockSpec((BLK, d2), lambda i: (i, 0))],
        out_shape=[jax.ShapeDtypeStruct((n, d2), jnp.float32),
                   jax.ShapeDtypeStruct((n, d2), jnp.float32)],
    )(degpT, accp, h1, b1r, W2p)


def _tc_final(degpT, accp, h2, b2r):
    n, d2 = h2.shape

    def body(dg_ref, ac_ref, h2_ref, b2_ref, o_ref):
        deg = 1.0 + dg_ref[:, 0:1] + dg_ref[:, 1:2]
        dinv = lax.rsqrt(deg)
        accsum = ac_ref[0] + ac_ref[1]
        o_ref[:] = dinv * accsum + (dinv * dinv) * h2_ref[:] + b2_ref[:]

    return pl.pallas_call(
        body,
        grid=(n // BLK,),
        in_specs=[pl.BlockSpec((BLK, 2), lambda i: (i, 0)),
                  pl.BlockSpec((2, BLK, d2), lambda i: (0, i, 0)),
                  pl.BlockSpec((BLK, d2), lambda i: (i, 0)),
                  pl.BlockSpec((1, d2), lambda i: (0, 0))],
        out_specs=pl.BlockSpec((BLK, d2), lambda i: (i, 0)),
        out_shape=jax.ShapeDtypeStruct((n, d2), jnp.float32),
    )(degpT, accp, h2, b2r)


def kernel(x, edge_index, edge_weight, W1, b1, W2, b2):
    n, nfeat = x.shape
    e = edge_weight.shape[0]
    nclass = W2.shape[1]
    d2 = ((nclass + L - 1) // L) * L

    n_pad = ((n + NS * B - 1) // (NS * B)) * (NS * B)
    e_pad = ((e + NW * B * 8 - 1) // (NW * B * 8)) * (NW * B * 8)

    src = edge_index[0]
    dst = edge_index[1]
    pad_e = e_pad - e
    src_p = jnp.concatenate([src, jnp.zeros((pad_e,), src.dtype)])
    dst_p = jnp.concatenate([dst, jnp.zeros((pad_e,), dst.dtype)])
    ew_p = jnp.concatenate(
        [edge_weight, jnp.zeros((pad_e,), edge_weight.dtype)])
    dst2d = dst_p.reshape(e_pad // B, B)
    ew2d = ew_p.reshape(e_pad // B, B)
    eidx = jnp.stack(
        [src_p.reshape(e_pad // BAG, BAG),
         dst_p.reshape(e_pad // BAG, BAG)], axis=1)
    ewg = ew_p.reshape(e_pad // BAG, 1, BAG)

    xp = jnp.pad(x, ((0, n_pad - n), (0, 0)))
    W2p = jnp.pad(W2, ((0, 0), (0, d2 - nclass)))
    b1r = b1.reshape(1, nfeat)
    b2r = jnp.pad(b2, (0, d2 - nclass)).reshape(1, d2)

    degp = _sc_deg(dst2d, ew2d, n_pad)
    degpT = degp.T

    h1 = _tc_matmul(xp, W1)
    hs1 = _tc_prescale(degpT, h1)
    acc1p = _sc_agg(eidx, ewg, hs1, n_pad, nfeat, c0=_C0)
    h2, hs2 = _tc_mid(degpT, acc1p, h1, b1r, W2p)
    acc2p = _sc_agg(eidx, ewg, hs2, n_pad, d2, c0=_C0)
    out = _tc_final(degpT, acc2p, h2, b2r)
    return out[:n, :nclass]


# speedup vs baseline: 1.0836x; 1.0836x over previous
import functools

import jax
import jax.numpy as jnp
from jax import lax
from jax.experimental import pallas as pl
from jax.experimental.pallas import tpu as pltpu
from jax.experimental.pallas import tpu_sc as plsc

NC, NS, L = 2, 16, 16
NW = NC * NS
B = 128
BAG = 64
BLK = 1024
_C0 = 288


def _mesh():
    return plsc.VectorSubcoreMesh(core_axis_name="c", subcore_axis_name="s")


def _sc_deg(dst2d, ew2d, n_pad):
    chunks = dst2d.shape[0]
    cpt = chunks // NW
    rpt = n_pad // NS

    @functools.partial(
        pl.kernel,
        out_type=jax.ShapeDtypeStruct((NC, n_pad), jnp.float32),
        mesh=_mesh(),
        scratch_types=[
            pltpu.VMEM((cpt, B), jnp.int32),
            pltpu.VMEM((cpt, B), jnp.float32),
            pltpu.VMEM((rpt,), jnp.float32),
            pltpu.VMEM_SHARED((n_pad,), jnp.float32),
        ],
    )
    def k(dst_hbm, ew_hbm, degp_hbm, dstb, ewb, zbuf, deg_sh):
        cid = lax.axis_index("c")
        sid = lax.axis_index("s")
        wid = cid * NS + sid

        def zb(i, carry):
            zbuf[pl.ds(i * L, L)] = jnp.zeros((L,), jnp.float32)
            return carry

        lax.fori_loop(0, rpt // L, zb, 0)
        pltpu.sync_copy(zbuf, deg_sh.at[pl.ds(sid * rpt, rpt)])
        pltpu.sync_copy(dst_hbm.at[pl.ds(wid * cpt, cpt)], dstb)
        pltpu.sync_copy(ew_hbm.at[pl.ds(wid * cpt, cpt)], ewb)
        plsc.subcore_barrier()

        def body(c, carry):
            pltpu.sync_copy(ewb.at[c], deg_sh.at[dstb.at[c]], add=True)
            return carry

        lax.fori_loop(0, cpt, body, 0)
        plsc.subcore_barrier()
        pltpu.sync_copy(deg_sh.at[pl.ds(sid * rpt, rpt)],
                        degp_hbm.at[cid, pl.ds(sid * rpt, rpt)])

    return k(dst2d, ew2d)


def _sc_agg(eidx, ewg, feat, n_pad, d, c0=None):
    chunks = eidx.shape[0]
    cpt = chunks // NW
    if c0 is None:
        c0 = cpt
    c1 = 2 * cpt - c0
    assert c0 % 4 == 0 and c1 % 4 == 0 and min(c0, c1) >= 8
    rpt = n_pad // NS
    nv = d // L

    @functools.partial(
        pl.kernel,
        out_type=jax.ShapeDtypeStruct((NC, n_pad, d), jnp.float32),
        mesh=_mesh(),
        compiler_params=pltpu.CompilerParams(use_tc_tiling_on_sc=False),
        scratch_types=[
            pltpu.VMEM((BAG, d), jnp.float32),
            pltpu.VMEM((BAG, d), jnp.float32),
            pltpu.VMEM((BAG, d), jnp.float32),
            pltpu.VMEM((BAG, d), jnp.float32),
            pltpu.VMEM((2, BAG), jnp.int32),
            pltpu.VMEM((2, BAG), jnp.int32),
            pltpu.VMEM((2, BAG), jnp.int32),
            pltpu.VMEM((2, BAG), jnp.int32),
            pltpu.VMEM((1, BAG), jnp.float32),
            pltpu.VMEM((1, BAG), jnp.float32),
            pltpu.VMEM((1, BAG), jnp.float32),
            pltpu.VMEM((1, BAG), jnp.float32),
            pltpu.VMEM_SHARED((n_pad, d), jnp.float32),
            pltpu.SemaphoreType.DMA,
            pltpu.SemaphoreType.DMA,
            pltpu.SemaphoreType.DMA,
            pltpu.SemaphoreType.DMA,
            pltpu.SemaphoreType.DMA,
            pltpu.SemaphoreType.DMA,
            pltpu.SemaphoreType.DMA,
            pltpu.SemaphoreType.DMA,
            pltpu.SemaphoreType.DMA,
            pltpu.SemaphoreType.DMA,
            pltpu.SemaphoreType.DMA,
            pltpu.SemaphoreType.DMA,
        ],
    )
    def k(ei_hbm, ew_hbm, feat_hbm, accp_hbm,
          rb0, rb1, sb0, sb1, ei0, ei1, ei2, ei3, ef0, ef1, ef2, ef3,
          acc_sh, gsem0, gsem1, ssem0, ssem1,
          isem0, isem1, isem2, isem3, fsem0, fsem1, fsem2, fsem3):
        cid = lax.axis_index("c")
        sid = lax.axis_index("s")
        base = jnp.where(cid == 0, sid * c0, NS * c0 + sid * c1)
        cnt = jnp.where(cid == 0, c0, c1)
        rbufs, sbufs = (rb0, rb1), (sb0, sb1)
        eibufs = (ei0, ei1, ei2, ei3)
        efbufs = (ef0, ef1, ef2, ef3)
        gsems, ssems = (gsem0, gsem1), (ssem0, ssem1)
        isems = (isem0, isem1, isem2, isem3)
        fsems = (fsem0, fsem1, fsem2, fsem3)

        def zr(r, carry):
            for j in range(nv):
                sb0[r, pl.ds(j * L, L)] = jnp.zeros((L,), jnp.float32)
            return carry

        lax.fori_loop(0, BAG, zr, 0)
        for j in range(rpt // BAG):
            pltpu.sync_copy(sb0, acc_sh.at[pl.ds(sid * rpt + j * BAG, BAG)])
        pltpu.sync_copy(ei_hbm.at[base], ei0)
        pltpu.sync_copy(ew_hbm.at[base], ef0)
        pltpu.async_copy(feat_hbm.at[ei0.at[0]], rb0, gsem0)
        pltpu.async_copy(ei_hbm.at[base + 1], ei1, isem1)
        pltpu.async_copy(ew_hbm.at[base + 1], ef1, fsem1)
        plsc.subcore_barrier()

        def body(g, carry):
            for b in range(4):
                c = 4 * g + b
                rb, sb = rbufs[b % 2], sbufs[b % 2]
                gsem, ssem = gsems[b % 2], ssems[b % 2]
                ein = eibufs[(b + 1) % 4]

                @pl.when(c + 1 < cnt)
                def _():
                    pltpu.make_async_copy(
                        ei_hbm.at[base + c + 1], ein, isems[(b + 1) % 4]
                    ).wait()
                    pltpu.async_copy(feat_hbm.at[ein.at[0]],
                                     rbufs[(b + 1) % 2], gsems[(b + 1) % 2])

                if b >= 2:
                    pltpu.make_async_copy(
                        sb, acc_sh.at[eibufs[b].at[1]], ssem).wait()
                else:
                    @pl.when(g > 0)
                    def _():
                        pltpu.make_async_copy(
                            sb, acc_sh.at[eibufs[b].at[1]], ssem).wait()

                @pl.when(c + 2 < cnt)
                def _():
                    pltpu.async_copy(ei_hbm.at[base + c + 2],
                                     eibufs[(b + 2) % 4], isems[(b + 2) % 4])
                    pltpu.async_copy(ew_hbm.at[base + c + 2],
                                     efbufs[(b + 2) % 4], fsems[(b + 2) % 4])

                pltpu.make_async_copy(
                    feat_hbm.at[eibufs[b].at[0]], rb, gsem).wait()
                if b != 0:
                    pltpu.make_async_copy(
                        ew_hbm.at[base + c], efbufs[b], fsems[b]).wait()
                else:
                    @pl.when(g > 0)
                    def _():
                        pltpu.make_async_copy(
                            ew_hbm.at[base + c], efbufs[b], fsems[b]).wait()

                def scale(gg, rc):
                    wv = efbufs[b][0, pl.ds(gg * L, L)]
                    for r in range(L):
                        w = wv[r]
                        row = gg * L + r
                        for j in range(nv):
                            sl = pl.ds(j * L, L)
                            sb[row, sl] = rb[row, sl] * w
                    return rc

                lax.fori_loop(0, BAG // L, scale, 0)
                pltpu.async_copy(sb, acc_sh.at[eibufs[b].at[1]], ssem,
                                 add=True)
            return carry

        lax.fori_loop(0, cnt // 4, body, 0)
        for b in range(2):
            pltpu.make_async_copy(
                sbufs[b], acc_sh.at[eibufs[b].at[1]], ssems[b]).wait()
        plsc.subcore_barrier()
        pltpu.sync_copy(acc_sh.at[pl.ds(sid * rpt, rpt)],
                        accp_hbm.at[cid, pl.ds(sid * rpt, rpt)])

    return k(eidx, ewg, feat)


def _tc_matmul(x, W):
    n, kdim = x.shape
    m = W.shape[1]

    def body(x_ref, w_ref, o_ref):
        o_ref[:] = jnp.dot(x_ref[:], w_ref[:],
                           preferred_element_type=jnp.float32)

    return pl.pallas_call(
        body,
        grid=(n // BLK,),
        in_specs=[pl.BlockSpec((BLK, kdim), lambda i: (i, 0)),
                  pl.BlockSpec((kdim, m), lambda i: (0, 0))],
        out_specs=pl.BlockSpec((BLK, m), lambda i: (i, 0)),
        out_shape=jax.ShapeDtypeStruct((n, m), jnp.float32),
    )(x, W)


def _tc_prescale(degpT, h):
    n, d = h.shape

    def body(dg_ref, h_ref, o_ref):
        deg = 1.0 + dg_ref[:, 0:1] + dg_ref[:, 1:2]
        dinv = lax.rsqrt(deg)
        o_ref[:] = h_ref[:] * dinv

    return pl.pallas_call(
        body,
        grid=(n // BLK,),
        in_specs=[pl.BlockSpec((BLK, 2), lambda i: (i, 0)),
                  pl.BlockSpec((BLK, d), lambda i: (i, 0))],
        out_specs=pl.BlockSpec((BLK, d), lambda i: (i, 0)),
        out_shape=jax.ShapeDtypeStruct((n, d), jnp.float32),
    )(degpT, h)


def _tc_mid(degpT, accp, h1, b1r, W2p):
    n, d1 = h1.shape
    d2 = W2p.shape[1]

    def body(dg_ref, ac_ref, h1_ref, b1_ref, w2_ref, h2_ref, hs2_ref):
        deg = 1.0 + dg_ref[:, 0:1] + dg_ref[:, 1:2]
        dinv = lax.rsqrt(deg)
        accsum = ac_ref[0] + ac_ref[1]
        x1 = dinv * accsum + (dinv * dinv) * h1_ref[:] + b1_ref[:]
        x1 = jnp.maximum(x1, 0.0)
        h2 = jnp.dot(x1, w2_ref[:], preferred_element_type=jnp.float32)
        h2_ref[:] = h2
        hs2_ref[:] = h2 * dinv

    return pl.pallas_call(
        body,
        grid=(n // BLK,),
        in_specs=[pl.BlockSpec((BLK, 2), lambda i: (i, 0)),
                  pl.BlockSpec((2, BLK, d1), lambda i: (0, i, 0)),
                  pl.BlockSpec((BLK, d1), lambda i: (i, 0)),
                  pl.BlockSpec((1, d1), lambda i: (0, 0)),
                  pl.BlockSpec((d1, d2), lambda i: (0, 0))],
        out_specs=[pl.BlockSpec((BLK, d2), lambda i: (i, 0)),
                   pl.BlockSpec((BLK, d2), lambda i: (i, 0))],
        out_shape=[jax.ShapeDtypeStruct((n, d2), jnp.float32),
                   jax.ShapeDtypeStruct((n, d2), jnp.float32)],
    )(degpT, accp, h1, b1r, W2p)


def _tc_final(degpT, accp, h2, b2r):
    n, d2 = h2.shape

    def body(dg_ref, ac_ref, h2_ref, b2_ref, o_ref):
        deg = 1.0 + dg_ref[:, 0:1] + dg_ref[:, 1:2]
        dinv = lax.rsqrt(deg)
        accsum = ac_ref[0] + ac_ref[1]
        o_ref[:] = dinv * accsum + (dinv * dinv) * h2_ref[:] + b2_ref[:]

    return pl.pallas_call(
        body,
        grid=(n // BLK,),
        in_specs=[pl.BlockSpec((BLK, 2), lambda i: (i, 0)),
                  pl.BlockSpec((2, BLK, d2), lambda i: (0, i, 0)),
                  pl.BlockSpec((BLK, d2), lambda i: (i, 0)),
                  pl.BlockSpec((1, d2), lambda i: (0, 0))],
        out_specs=pl.BlockSpec((BLK, d2), lambda i: (i, 0)),
        out_shape=jax.ShapeDtypeStruct((n, d2), jnp.float32),
    )(degpT, accp, h2, b2r)


def kernel(x, edge_index, edge_weight, W1, b1, W2, b2):
    n, nfeat = x.shape
    e = edge_weight.shape[0]
    nclass = W2.shape[1]
    d2 = ((nclass + L - 1) // L) * L

    n_pad = ((n + NS * B - 1) // (NS * B)) * (NS * B)
    e_pad = ((e + NW * B * 8 - 1) // (NW * B * 8)) * (NW * B * 8)

    src = edge_index[0]
    dst = edge_index[1]
    pad_e = e_pad - e
    src_p = jnp.concatenate([src, jnp.zeros((pad_e,), src.dtype)])
    dst_p = jnp.concatenate([dst, jnp.zeros((pad_e,), dst.dtype)])
    ew_p = jnp.concatenate(
        [edge_weight, jnp.zeros((pad_e,), edge_weight.dtype)])
    dst2d = dst_p.reshape(e_pad // B, B)
    ew2d = ew_p.reshape(e_pad // B, B)
    eidx = jnp.stack(
        [src_p.reshape(e_pad // BAG, BAG),
         dst_p.reshape(e_pad // BAG, BAG)], axis=1)
    ewg = ew_p.reshape(e_pad // BAG, 1, BAG)

    xp = jnp.pad(x, ((0, n_pad - n), (0, 0)))
    W2p = jnp.pad(W2, ((0, 0), (0, d2 - nclass)))
    b1r = b1.reshape(1, nfeat)
    b2r = jnp.pad(b2, (0, d2 - nclass)).reshape(1, d2)

    degp = _sc_deg(dst2d, ew2d, n_pad)
    degpT = degp.T

    h1 = _tc_matmul(xp, W1)
    hs1 = _tc_prescale(degpT, h1)
    acc1p = _sc_agg(eidx, ewg, hs1, n_pad, nfeat, c0=_C0)
    h2, hs2 = _tc_mid(degpT, acc1p, h1, b1r, W2p)
    acc2p = _sc_agg(eidx, ewg, hs2, n_pad, d2, c0=_C0)
    out = _tc_final(degpT, acc2p, h2, b2r)
    return out[:n, :nclass]


# speedup vs baseline: 1.1576x; 1.0683x over previous
import functools

import jax
import jax.numpy as jnp
from jax import lax
from jax.experimental import pallas as pl
from jax.experimental.pallas import tpu as pltpu
from jax.experimental.pallas import tpu_sc as plsc

NC, NS, L = 2, 16, 16
NW = NC * NS
B = 128
BAG = 64
BLK = 1024
_C0 = 272


def _mesh():
    return plsc.VectorSubcoreMesh(core_axis_name="c", subcore_axis_name="s")


def _sc_deg(dst2d, ew2d, n_pad):
    chunks = dst2d.shape[0]
    cpt = chunks // NW
    rpt = n_pad // NS

    @functools.partial(
        pl.kernel,
        out_type=jax.ShapeDtypeStruct((NC, n_pad), jnp.float32),
        mesh=_mesh(),
        scratch_types=[
            pltpu.VMEM((cpt, B), jnp.int32),
            pltpu.VMEM((cpt, B), jnp.float32),
            pltpu.VMEM((rpt,), jnp.float32),
            pltpu.VMEM_SHARED((n_pad,), jnp.float32),
        ],
    )
    def k(dst_hbm, ew_hbm, degp_hbm, dstb, ewb, zbuf, deg_sh):
        cid = lax.axis_index("c")
        sid = lax.axis_index("s")
        wid = cid * NS + sid

        def zb(i, carry):
            zbuf[pl.ds(i * L, L)] = jnp.zeros((L,), jnp.float32)
            return carry

        lax.fori_loop(0, rpt // L, zb, 0)
        pltpu.sync_copy(zbuf, deg_sh.at[pl.ds(sid * rpt, rpt)])
        pltpu.sync_copy(dst_hbm.at[pl.ds(wid * cpt, cpt)], dstb)
        pltpu.sync_copy(ew_hbm.at[pl.ds(wid * cpt, cpt)], ewb)
        plsc.subcore_barrier()

        def body(c, carry):
            pltpu.sync_copy(ewb.at[c], deg_sh.at[dstb.at[c]], add=True)
            return carry

        lax.fori_loop(0, cpt, body, 0)
        plsc.subcore_barrier()
        pltpu.sync_copy(deg_sh.at[pl.ds(sid * rpt, rpt)],
                        degp_hbm.at[cid, pl.ds(sid * rpt, rpt)])

    return k(dst2d, ew2d)


def _sc_agg(eidx, ewg, feat, n_pad, d, c0=None):
    chunks = eidx.shape[0]
    cpt = chunks // NW
    if c0 is None:
        c0 = cpt
    c1 = 2 * cpt - c0
    assert c0 % 4 == 0 and c1 % 4 == 0 and min(c0, c1) >= 8
    rpt = n_pad // NS
    nv = d // L

    @functools.partial(
        pl.kernel,
        out_type=jax.ShapeDtypeStruct((NC, n_pad, d), jnp.float32),
        mesh=_mesh(),
        compiler_params=pltpu.CompilerParams(use_tc_tiling_on_sc=False),
        scratch_types=[
            pltpu.VMEM((BAG, d), jnp.float32),
            pltpu.VMEM((BAG, d), jnp.float32),
            pltpu.VMEM((BAG, d), jnp.float32),
            pltpu.VMEM((BAG, d), jnp.float32),
            pltpu.VMEM((2, BAG), jnp.int32),
            pltpu.VMEM((2, BAG), jnp.int32),
            pltpu.VMEM((2, BAG), jnp.int32),
            pltpu.VMEM((2, BAG), jnp.int32),
            pltpu.VMEM((1, BAG), jnp.float32),
            pltpu.VMEM((1, BAG), jnp.float32),
            pltpu.VMEM((1, BAG), jnp.float32),
            pltpu.VMEM((1, BAG), jnp.float32),
            pltpu.VMEM_SHARED((n_pad, d), jnp.float32),
            pltpu.SemaphoreType.DMA,
            pltpu.SemaphoreType.DMA,
            pltpu.SemaphoreType.DMA,
            pltpu.SemaphoreType.DMA,
            pltpu.SemaphoreType.DMA,
            pltpu.SemaphoreType.DMA,
            pltpu.SemaphoreType.DMA,
            pltpu.SemaphoreType.DMA,
            pltpu.SemaphoreType.DMA,
            pltpu.SemaphoreType.DMA,
            pltpu.SemaphoreType.DMA,
            pltpu.SemaphoreType.DMA,
        ],
    )
    def k(ei_hbm, ew_hbm, feat_hbm, accp_hbm,
          rb0, rb1, sb0, sb1, ei0, ei1, ei2, ei3, ef0, ef1, ef2, ef3,
          acc_sh, gsem0, gsem1, ssem0, ssem1,
          isem0, isem1, isem2, isem3, fsem0, fsem1, fsem2, fsem3):
        cid = lax.axis_index("c")
        sid = lax.axis_index("s")
        base = jnp.where(cid == 0, sid * c0, NS * c0 + sid * c1)
        cnt = jnp.where(cid == 0, c0, c1)
        rbufs, sbufs = (rb0, rb1), (sb0, sb1)
        eibufs = (ei0, ei1, ei2, ei3)
        efbufs = (ef0, ef1, ef2, ef3)
        gsems, ssems = (gsem0, gsem1), (ssem0, ssem1)
        isems = (isem0, isem1, isem2, isem3)
        fsems = (fsem0, fsem1, fsem2, fsem3)

        def zr(r, carry):
            for j in range(nv):
                sb0[r, pl.ds(j * L, L)] = jnp.zeros((L,), jnp.float32)
            return carry

        lax.fori_loop(0, BAG, zr, 0)
        for j in range(rpt // BAG):
            pltpu.sync_copy(sb0, acc_sh.at[pl.ds(sid * rpt + j * BAG, BAG)])
        pltpu.sync_copy(ei_hbm.at[base], ei0)
        pltpu.sync_copy(ew_hbm.at[base], ef0)
        pltpu.async_copy(feat_hbm.at[ei0.at[0]], rb0, gsem0)
        pltpu.async_copy(ei_hbm.at[base + 1], ei1, isem1)
        pltpu.async_copy(ew_hbm.at[base + 1], ef1, fsem1)
        plsc.subcore_barrier()

        def body(g, carry):
            for b in range(4):
                c = 4 * g + b
                rb, sb = rbufs[b % 2], sbufs[b % 2]
                gsem, ssem = gsems[b % 2], ssems[b % 2]
                ein = eibufs[(b + 1) % 4]

                @pl.when(c + 1 < cnt)
                def _():
                    pltpu.make_async_copy(
                        ei_hbm.at[base + c + 1], ein, isems[(b + 1) % 4]
                    ).wait()
                    pltpu.async_copy(feat_hbm.at[ein.at[0]],
                                     rbufs[(b + 1) % 2], gsems[(b + 1) % 2])

                if b >= 2:
                    pltpu.make_async_copy(
                        sb, acc_sh.at[eibufs[b].at[1]], ssem).wait()
                else:
                    @pl.when(g > 0)
                    def _():
                        pltpu.make_async_copy(
                            sb, acc_sh.at[eibufs[b].at[1]], ssem).wait()

                @pl.when(c + 2 < cnt)
                def _():
                    pltpu.async_copy(ei_hbm.at[base + c + 2],
                                     eibufs[(b + 2) % 4], isems[(b + 2) % 4])
                    pltpu.async_copy(ew_hbm.at[base + c + 2],
                                     efbufs[(b + 2) % 4], fsems[(b + 2) % 4])

                pltpu.make_async_copy(
                    feat_hbm.at[eibufs[b].at[0]], rb, gsem).wait()
                if b != 0:
                    pltpu.make_async_copy(
                        ew_hbm.at[base + c], efbufs[b], fsems[b]).wait()
                else:
                    @pl.when(g > 0)
                    def _():
                        pltpu.make_async_copy(
                            ew_hbm.at[base + c], efbufs[b], fsems[b]).wait()

                def scale(gg, rc):
                    wv = efbufs[b][0, pl.ds(gg * L, L)]
                    for r in range(L):
                        w = wv[r]
                        row = gg * L + r
                        for j in range(nv):
                            sl = pl.ds(j * L, L)
                            sb[row, sl] = rb[row, sl] * w
                    return rc

                lax.fori_loop(0, BAG // L, scale, 0)
                pltpu.async_copy(sb, acc_sh.at[eibufs[b].at[1]], ssem,
                                 add=True)
            return carry

        lax.fori_loop(0, cnt // 4, body, 0)
        for b in range(2):
            pltpu.make_async_copy(
                sbufs[b], acc_sh.at[eibufs[b].at[1]], ssems[b]).wait()
        plsc.subcore_barrier()
        pltpu.sync_copy(acc_sh.at[pl.ds(sid * rpt, rpt)],
                        accp_hbm.at[cid, pl.ds(sid * rpt, rpt)])

    return k(eidx, ewg, feat)


def _tc_mm_scale(x, W, degpT):
    n, kdim = x.shape
    m = W.shape[1]

    def body(x_ref, w_ref, dg_ref, h_ref, hs_ref):
        deg = 1.0 + dg_ref[:, 0:1] + dg_ref[:, 1:2]
        dinv = lax.rsqrt(deg)
        h = jnp.dot(x_ref[:], w_ref[:], preferred_element_type=jnp.float32)
        h_ref[:] = h
        hs_ref[:] = h * dinv

    return pl.pallas_call(
        body,
        grid=(n // BLK,),
        in_specs=[pl.BlockSpec((BLK, kdim), lambda i: (i, 0)),
                  pl.BlockSpec((kdim, m), lambda i: (0, 0)),
                  pl.BlockSpec((BLK, 2), lambda i: (i, 0))],
        out_specs=[pl.BlockSpec((BLK, m), lambda i: (i, 0)),
                   pl.BlockSpec((BLK, m), lambda i: (i, 0))],
        out_shape=[jax.ShapeDtypeStruct((n, m), jnp.float32),
                   jax.ShapeDtypeStruct((n, m), jnp.float32)],
    )(x, W, degpT)


def _tc_mid(degpT, accp, h1, b1r, W2p):
    n, d1 = h1.shape
    d2 = W2p.shape[1]

    def body(dg_ref, ac_ref, h1_ref, b1_ref, w2_ref, h2_ref, hs2_ref):
        deg = 1.0 + dg_ref[:, 0:1] + dg_ref[:, 1:2]
        dinv = lax.rsqrt(deg)
        accsum = ac_ref[0] + ac_ref[1]
        x1 = dinv * accsum + (dinv * dinv) * h1_ref[:] + b1_ref[:]
        x1 = jnp.maximum(x1, 0.0)
        h2 = jnp.dot(x1, w2_ref[:], preferred_element_type=jnp.float32)
        h2_ref[:] = h2
        hs2_ref[:] = h2 * dinv

    return pl.pallas_call(
        body,
        grid=(n // BLK,),
        in_specs=[pl.BlockSpec((BLK, 2), lambda i: (i, 0)),
                  pl.BlockSpec((2, BLK, d1), lambda i: (0, i, 0)),
                  pl.BlockSpec((BLK, d1), lambda i: (i, 0)),
                  pl.BlockSpec((1, d1), lambda i: (0, 0)),
                  pl.BlockSpec((d1, d2), lambda i: (0, 0))],
        out_specs=[pl.BlockSpec((BLK, d2), lambda i: (i, 0)),
                   pl.BlockSpec((BLK, d2), lambda i: (i, 0))],
        out_shape=[jax.ShapeDtypeStruct((n, d2), jnp.float32),
                   jax.ShapeDtypeStruct((n, d2), jnp.float32)],
    )(degpT, accp, h1, b1r, W2p)


def _tc_final(degpT, accp, h2, b2r):
    n, d2 = h2.shape

    def body(dg_ref, ac_ref, h2_ref, b2_ref, o_ref):
        deg = 1.0 + dg_ref[:, 0:1] + dg_ref[:, 1:2]
        dinv = lax.rsqrt(deg)
        accsum = ac_ref[0] + ac_ref[1]
        o_ref[:] = dinv * accsum + (dinv * dinv) * h2_ref[:] + b2_ref[:]

    return pl.pallas_call(
        body,
        grid=(n // BLK,),
        in_specs=[pl.BlockSpec((BLK, 2), lambda i: (i, 0)),
                  pl.BlockSpec((2, BLK, d2), lambda i: (0, i, 0)),
                  pl.BlockSpec((BLK, d2), lambda i: (i, 0)),
                  pl.BlockSpec((1, d2), lambda i: (0, 0))],
        out_specs=pl.BlockSpec((BLK, d2), lambda i: (i, 0)),
        out_shape=jax.ShapeDtypeStruct((n, d2), jnp.float32),
    )(degpT, accp, h2, b2r)


def kernel(x, edge_index, edge_weight, W1, b1, W2, b2):
    n, nfeat = x.shape
    e = edge_weight.shape[0]
    nclass = W2.shape[1]
    d2 = ((nclass + L - 1) // L) * L

    n_pad = ((n + NS * B - 1) // (NS * B)) * (NS * B)
    e_pad = ((e + NW * B * 8 - 1) // (NW * B * 8)) * (NW * B * 8)

    src = edge_index[0]
    dst = edge_index[1]
    pad_e = e_pad - e
    src_p = jnp.concatenate([src, jnp.zeros((pad_e,), src.dtype)])
    dst_p = jnp.concatenate([dst, jnp.zeros((pad_e,), dst.dtype)])
    ew_p = jnp.concatenate(
        [edge_weight, jnp.zeros((pad_e,), edge_weight.dtype)])
    dst2d = dst_p.reshape(e_pad // B, B)
    ew2d = ew_p.reshape(e_pad // B, B)
    eidx = jnp.stack(
        [src_p.reshape(e_pad // BAG, BAG),
         dst_p.reshape(e_pad // BAG, BAG)], axis=1)
    ewg = ew_p.reshape(e_pad // BAG, 1, BAG)

    xp = jnp.pad(x, ((0, n_pad - n), (0, 0)))
    W2p = jnp.pad(W2, ((0, 0), (0, d2 - nclass)))
    b1r = b1.reshape(1, nfeat)
    b2r = jnp.pad(b2, (0, d2 - nclass)).reshape(1, d2)

    degp = _sc_deg(dst2d, ew2d, n_pad)
    degpT = degp.T

    h1, hs1 = _tc_mm_scale(xp, W1, degpT)
    acc1p = _sc_agg(eidx, ewg, hs1, n_pad, nfeat, c0=_C0)
    h2, hs2 = _tc_mid(degpT, acc1p, h1, b1r, W2p)
    acc2p = _sc_agg(eidx, ewg, hs2, n_pad, d2, c0=_C0)
    out = _tc_final(degpT, acc2p, h2, b2r)
    return out[:n, :nclass]


# speedup vs baseline: 1.1758x; 1.0157x over previous
import functools

import jax
import jax.numpy as jnp
from jax import lax
from jax.experimental import pallas as pl
from jax.experimental.pallas import tpu as pltpu
from jax.experimental.pallas import tpu_sc as plsc

NC, NS, L = 2, 16, 16
NW = NC * NS
B = 128
BAG = 64
BLK = 1024
_C0 = 272


def _mesh():
    return plsc.VectorSubcoreMesh(core_axis_name="c", subcore_axis_name="s")


def _sc_deg(dst2d, ew2d, n_pad):
    chunks = dst2d.shape[0]
    cpt = chunks // NW
    rpt = n_pad // NS

    @functools.partial(
        pl.kernel,
        out_type=jax.ShapeDtypeStruct((NC, n_pad), jnp.float32),
        mesh=_mesh(),
        scratch_types=[
            pltpu.VMEM((cpt, B), jnp.int32),
            pltpu.VMEM((cpt, B), jnp.float32),
            pltpu.VMEM((rpt,), jnp.float32),
            pltpu.VMEM_SHARED((n_pad,), jnp.float32),
        ],
    )
    def k(dst_hbm, ew_hbm, degp_hbm, dstb, ewb, zbuf, deg_sh):
        cid = lax.axis_index("c")
        sid = lax.axis_index("s")
        wid = cid * NS + sid

        def zb(i, carry):
            zbuf[pl.ds(i * L, L)] = jnp.zeros((L,), jnp.float32)
            return carry

        lax.fori_loop(0, rpt // L, zb, 0)
        pltpu.sync_copy(zbuf, deg_sh.at[pl.ds(sid * rpt, rpt)])
        pltpu.sync_copy(dst_hbm.at[pl.ds(wid * cpt, cpt)], dstb)
        pltpu.sync_copy(ew_hbm.at[pl.ds(wid * cpt, cpt)], ewb)
        plsc.subcore_barrier()

        def body(c, carry):
            pltpu.sync_copy(ewb.at[c], deg_sh.at[dstb.at[c]], add=True)
            return carry

        lax.fori_loop(0, cpt, body, 0)
        plsc.subcore_barrier()
        pltpu.sync_copy(deg_sh.at[pl.ds(sid * rpt, rpt)],
                        degp_hbm.at[cid, pl.ds(sid * rpt, rpt)])

    return k(dst2d, ew2d)


def _sc_agg(eidx, ewg, feat, n_pad, d, c0=None):
    chunks = eidx.shape[0]
    cpt = chunks // NW
    if c0 is None:
        c0 = cpt
    c1 = 2 * cpt - c0
    assert c0 % 4 == 0 and c1 % 4 == 0 and min(c0, c1) >= 8
    rpt = n_pad // NS
    nv = d // L

    @functools.partial(
        pl.kernel,
        out_type=jax.ShapeDtypeStruct((NC, n_pad, d), jnp.float32),
        mesh=_mesh(),
        compiler_params=pltpu.CompilerParams(use_tc_tiling_on_sc=False),
        scratch_types=[
            pltpu.VMEM((BAG, d), jnp.float32),
            pltpu.VMEM((BAG, d), jnp.float32),
            pltpu.VMEM((BAG, d), jnp.float32),
            pltpu.VMEM((BAG, d), jnp.float32),
            pltpu.VMEM((2, BAG), jnp.int32),
            pltpu.VMEM((2, BAG), jnp.int32),
            pltpu.VMEM((2, BAG), jnp.int32),
            pltpu.VMEM((2, BAG), jnp.int32),
            pltpu.VMEM((1, BAG), jnp.float32),
            pltpu.VMEM((1, BAG), jnp.float32),
            pltpu.VMEM((1, BAG), jnp.float32),
            pltpu.VMEM((1, BAG), jnp.float32),
            pltpu.VMEM_SHARED((n_pad, d), jnp.float32),
            pltpu.SemaphoreType.DMA,
            pltpu.SemaphoreType.DMA,
            pltpu.SemaphoreType.DMA,
            pltpu.SemaphoreType.DMA,
            pltpu.SemaphoreType.DMA,
            pltpu.SemaphoreType.DMA,
            pltpu.SemaphoreType.DMA,
            pltpu.SemaphoreType.DMA,
            pltpu.SemaphoreType.DMA,
            pltpu.SemaphoreType.DMA,
            pltpu.SemaphoreType.DMA,
            pltpu.SemaphoreType.DMA,
        ],
    )
    def k(ei_hbm, ew_hbm, feat_hbm, accp_hbm,
          rb0, rb1, sb0, sb1, ei0, ei1, ei2, ei3, ef0, ef1, ef2, ef3,
          acc_sh, gsem0, gsem1, ssem0, ssem1,
          isem0, isem1, isem2, isem3, fsem0, fsem1, fsem2, fsem3):
        cid = lax.axis_index("c")
        sid = lax.axis_index("s")
        base = jnp.where(cid == 0, sid * c0, NS * c0 + sid * c1)
        cnt = jnp.where(cid == 0, c0, c1)
        rbufs, sbufs = (rb0, rb1), (sb0, sb1)
        eibufs = (ei0, ei1, ei2, ei3)
        efbufs = (ef0, ef1, ef2, ef3)
        gsems, ssems = (gsem0, gsem1), (ssem0, ssem1)
        isems = (isem0, isem1, isem2, isem3)
        fsems = (fsem0, fsem1, fsem2, fsem3)

        def zr(r, carry):
            for j in range(nv):
                sb0[r, pl.ds(j * L, L)] = jnp.zeros((L,), jnp.float32)
            return carry

        lax.fori_loop(0, BAG, zr, 0)
        for j in range(rpt // BAG):
            pltpu.sync_copy(sb0, acc_sh.at[pl.ds(sid * rpt + j * BAG, BAG)])
        pltpu.sync_copy(ei_hbm.at[base], ei0)
        pltpu.sync_copy(ew_hbm.at[base], ef0)
        pltpu.async_copy(feat_hbm.at[ei0.at[0]], rb0, gsem0)
        pltpu.async_copy(ei_hbm.at[base + 1], ei1, isem1)
        pltpu.async_copy(ew_hbm.at[base + 1], ef1, fsem1)
        plsc.subcore_barrier()

        def body(g, carry):
            for b in range(4):
                c = 4 * g + b
                rb, sb = rbufs[b % 2], sbufs[b % 2]
                gsem, ssem = gsems[b % 2], ssems[b % 2]
                ein = eibufs[(b + 1) % 4]

                @pl.when(c + 1 < cnt)
                def _():
                    pltpu.make_async_copy(
                        ei_hbm.at[base + c + 1], ein, isems[(b + 1) % 4]
                    ).wait()
                    pltpu.async_copy(feat_hbm.at[ein.at[0]],
                                     rbufs[(b + 1) % 2], gsems[(b + 1) % 2])

                if b >= 2:
                    pltpu.make_async_copy(
                        sb, acc_sh.at[eibufs[b].at[1]], ssem).wait()
                else:
                    @pl.when(g > 0)
                    def _():
                        pltpu.make_async_copy(
                            sb, acc_sh.at[eibufs[b].at[1]], ssem).wait()

                @pl.when(c + 2 < cnt)
                def _():
                    pltpu.async_copy(ei_hbm.at[base + c + 2],
                                     eibufs[(b + 2) % 4], isems[(b + 2) % 4])
                    pltpu.async_copy(ew_hbm.at[base + c + 2],
                                     efbufs[(b + 2) % 4], fsems[(b + 2) % 4])

                pltpu.make_async_copy(
                    feat_hbm.at[eibufs[b].at[0]], rb, gsem).wait()
                if b != 0:
                    pltpu.make_async_copy(
                        ew_hbm.at[base + c], efbufs[b], fsems[b]).wait()
                else:
                    @pl.when(g > 0)
                    def _():
                        pltpu.make_async_copy(
                            ew_hbm.at[base + c], efbufs[b], fsems[b]).wait()

                def scale(gg, rc):
                    wv = efbufs[b][0, pl.ds(gg * L, L)]
                    for r in range(L):
                        w = wv[r]
                        row = gg * L + r
                        for j in range(nv):
                            sl = pl.ds(j * L, L)
                            sb[row, sl] = rb[row, sl] * w
                    return rc

                lax.fori_loop(0, BAG // L, scale, 0)
                pltpu.async_copy(sb, acc_sh.at[eibufs[b].at[1]], ssem,
                                 add=True)
            return carry

        lax.fori_loop(0, cnt // 4, body, 0)
        for b in range(2):
            pltpu.make_async_copy(
                sbufs[b], acc_sh.at[eibufs[b].at[1]], ssems[b]).wait()
        plsc.subcore_barrier()
        pltpu.sync_copy(acc_sh.at[pl.ds(sid * rpt, rpt)],
                        accp_hbm.at[cid, pl.ds(sid * rpt, rpt)])

    return k(eidx, ewg, feat)


def _tc_matmul(x, W):
    n, kdim = x.shape
    m = W.shape[1]

    def body(x_ref, w_ref, o_ref):
        o_ref[:] = jnp.dot(x_ref[:], w_ref[:],
                           preferred_element_type=jnp.float32)

    return pl.pallas_call(
        body,
        grid=(n // BLK,),
        in_specs=[pl.BlockSpec((BLK, kdim), lambda i: (i, 0)),
                  pl.BlockSpec((kdim, m), lambda i: (0, 0))],
        out_specs=pl.BlockSpec((BLK, m), lambda i: (i, 0)),
        out_shape=jax.ShapeDtypeStruct((n, m), jnp.float32),
    )(x, W)


def _tc_prescale(degpT, h):
    n, d = h.shape

    def body(dg_ref, h_ref, o_ref):
        deg = 1.0 + dg_ref[:, 0:1] + dg_ref[:, 1:2]
        dinv = lax.rsqrt(deg)
        o_ref[:] = h_ref[:] * dinv

    return pl.pallas_call(
        body,
        grid=(n // BLK,),
        in_specs=[pl.BlockSpec((BLK, 2), lambda i: (i, 0)),
                  pl.BlockSpec((BLK, d), lambda i: (i, 0))],
        out_specs=pl.BlockSpec((BLK, d), lambda i: (i, 0)),
        out_shape=jax.ShapeDtypeStruct((n, d), jnp.float32),
    )(degpT, h)


def _tc_mid(degpT, accp, h1, b1r, W2p):
    n, d1 = h1.shape
    d2 = W2p.shape[1]

    def body(dg_ref, ac_ref, h1_ref, b1_ref, w2_ref, h2_ref, hs2_ref):
        deg = 1.0 + dg_ref[:, 0:1] + dg_ref[:, 1:2]
        dinv = lax.rsqrt(deg)
        accsum = ac_ref[0] + ac_ref[1]
        x1 = dinv * accsum + (dinv * dinv) * h1_ref[:] + b1_ref[:]
        x1 = jnp.maximum(x1, 0.0)
        h2 = jnp.dot(x1, w2_ref[:], preferred_element_type=jnp.float32)
        h2_ref[:] = h2
        hs2_ref[:] = h2 * dinv

    return pl.pallas_call(
        body,
        grid=(n // BLK,),
        in_specs=[pl.BlockSpec((BLK, 2), lambda i: (i, 0)),
                  pl.BlockSpec((2, BLK, d1), lambda i: (0, i, 0)),
                  pl.BlockSpec((BLK, d1), lambda i: (i, 0)),
                  pl.BlockSpec((1, d1), lambda i: (0, 0)),
                  pl.BlockSpec((d1, d2), lambda i: (0, 0))],
        out_specs=[pl.BlockSpec((BLK, d2), lambda i: (i, 0)),
                   pl.BlockSpec((BLK, d2), lambda i: (i, 0))],
        out_shape=[jax.ShapeDtypeStruct((n, d2), jnp.float32),
                   jax.ShapeDtypeStruct((n, d2), jnp.float32)],
    )(degpT, accp, h1, b1r, W2p)


def _tc_final(degpT, accp, h2, b2r):
    n, d2 = h2.shape

    def body(dg_ref, ac_ref, h2_ref, b2_ref, o_ref):
        deg = 1.0 + dg_ref[:, 0:1] + dg_ref[:, 1:2]
        dinv = lax.rsqrt(deg)
        accsum = ac_ref[0] + ac_ref[1]
        o_ref[:] = dinv * accsum + (dinv * dinv) * h2_ref[:] + b2_ref[:]

    return pl.pallas_call(
        body,
        grid=(n // BLK,),
        in_specs=[pl.BlockSpec((BLK, 2), lambda i: (i, 0)),
                  pl.BlockSpec((2, BLK, d2), lambda i: (0, i, 0)),
                  pl.BlockSpec((BLK, d2), lambda i: (i, 0)),
                  pl.BlockSpec((1, d2), lambda i: (0, 0))],
        out_specs=pl.BlockSpec((BLK, d2), lambda i: (i, 0)),
        out_shape=jax.ShapeDtypeStruct((n, d2), jnp.float32),
    )(degpT, accp, h2, b2r)


def kernel(x, edge_index, edge_weight, W1, b1, W2, b2):
    n, nfeat = x.shape
    e = edge_weight.shape[0]
    nclass = W2.shape[1]
    d2 = ((nclass + L - 1) // L) * L

    n_pad = ((n + NS * B - 1) // (NS * B)) * (NS * B)
    e_pad = ((e + NW * B * 8 - 1) // (NW * B * 8)) * (NW * B * 8)

    src = edge_index[0]
    dst = edge_index[1]
    pad_e = e_pad - e
    src_p = jnp.concatenate([src, jnp.zeros((pad_e,), src.dtype)])
    dst_p = jnp.concatenate([dst, jnp.zeros((pad_e,), dst.dtype)])
    ew_p = jnp.concatenate(
        [edge_weight, jnp.zeros((pad_e,), edge_weight.dtype)])
    dst2d = dst_p.reshape(e_pad // B, B)
    ew2d = ew_p.reshape(e_pad // B, B)
    eidx = jnp.stack(
        [src_p.reshape(e_pad // BAG, BAG),
         dst_p.reshape(e_pad // BAG, BAG)], axis=1)
    ewg = ew_p.reshape(e_pad // BAG, 1, BAG)

    xp = jnp.pad(x, ((0, n_pad - n), (0, 0)))
    W2p = jnp.pad(W2, ((0, 0), (0, d2 - nclass)))
    b1r = b1.reshape(1, nfeat)
    b2r = jnp.pad(b2, (0, d2 - nclass)).reshape(1, d2)

    degp = _sc_deg(dst2d, ew2d, n_pad)
    degpT = degp.T

    h1 = _tc_matmul(xp, W1)
    hs1 = _tc_prescale(degpT, h1)
    acc1p = _sc_agg(eidx, ewg, hs1, n_pad, nfeat, c0=_C0)
    h2, hs2 = _tc_mid(degpT, acc1p, h1, b1r, W2p)
    acc2p = _sc_agg(eidx, ewg, hs2, n_pad, d2, c0=_C0)
    out = _tc_final(degpT, acc2p, h2, b2r)
    return out[:n, :nclass]
